# Initial kernel scaffold; baseline (speedup 1.0000x reference)
import functools

import jax
import jax.numpy as jnp
from jax import lax
from jax.experimental import pallas as pl
from jax.experimental.pallas import tpu as pltpu
from jax.experimental.pallas import tpu_sc as plsc

_DIM = 32
_N_NODES = 100000
_N_EDGES = 1600000
_N_GRAPHS = 256
_EPS = 1e-5

_NC = 2
_NS = 16
_NW = _NC * _NS
_CH = 2000
_E_PER_W = _N_EDGES // _NW
_E_CHUNKS = _E_PER_W // _CH
_NODE_CHUNKS = _N_NODES // _CH


def _sc_segment_sums(src, edge_attr, batch, x, zeros):
    mesh = plsc.VectorSubcoreMesh(core_axis_name="c", subcore_axis_name="s")

    @functools.partial(
        pl.kernel,
        mesh=mesh,
        out_type=[
            jax.ShapeDtypeStruct((_NC, 2, _N_GRAPHS, _DIM), jnp.float32),
            jax.ShapeDtypeStruct((_NW, 2, _N_GRAPHS), jnp.float32),
        ],
        scratch_types=[
            pltpu.VMEM((_CH,), jnp.int32),
            pltpu.VMEM((_CH,), jnp.int32),
            pltpu.VMEM((_CH, _DIM), jnp.float32),
            pltpu.VMEM((_N_GRAPHS,), jnp.float32),
            pltpu.VMEM((_N_GRAPHS,), jnp.float32),
            pltpu.VMEM_SHARED((_N_GRAPHS, _DIM), jnp.float32),
            pltpu.VMEM_SHARED((_N_GRAPHS, _DIM), jnp.float32),
            pltpu.SemaphoreType.DMA,
        ],
    )
    def k(src_hbm, attr_hbm, batch_hbm, x_hbm, zeros_hbm,
          out_sums, out_cnts,
          src_v, g_v, rows_v, cnt_e, cnt_v, acc_e, acc_v, sem):
        cid = lax.axis_index("c")
        sid = lax.axis_index("s")
        w = cid * _NS + sid

        @pl.when(sid == 0)
        def _():
            pltpu.sync_copy(zeros_hbm, acc_e)
            pltpu.sync_copy(zeros_hbm, acc_v)

        zero16 = jnp.zeros((16,), jnp.float32)

        def zinit(i, carry):
            cnt_e[pl.ds(i * 16, 16)] = zero16
            cnt_v[pl.ds(i * 16, 16)] = zero16
            return carry

        lax.fori_loop(0, _N_GRAPHS // 16, zinit, 0)

        plsc.subcore_barrier()

        ones16 = jnp.ones((16,), jnp.float32)

        def hist(cref):
            def body(j, carry):
                gv = g_v[pl.ds(j * 16, 16)]
                plsc.addupdate_scatter(cref, [gv], ones16)
                return carry
            lax.fori_loop(0, _CH // 16, body, 0)

        ebase = w * _E_PER_W

        def echunk(c, carry):
            off = ebase + c * _CH
            pltpu.sync_copy(src_hbm.at[pl.ds(off, _CH)], src_v)
            pltpu.async_copy(batch_hbm.at[src_v], g_v, sem).wait()
            pltpu.sync_copy(attr_hbm.at[pl.ds(off, _CH)], rows_v)
            pltpu.sync_copy(rows_v, acc_e.at[g_v], add=True)
            hist(cnt_e)
            return carry

        lax.fori_loop(0, _E_CHUNKS, echunk, 0)

        def nchunk(chunk):
            off = chunk * _CH
            pltpu.sync_copy(batch_hbm.at[pl.ds(off, _CH)], g_v)
            pltpu.sync_copy(x_hbm.at[pl.ds(off, _CH)], rows_v)
            pltpu.sync_copy(rows_v, acc_v.at[g_v], add=True)
            hist(cnt_v)

        nchunk(w)

        @pl.when(w + _NW < _NODE_CHUNKS)
        def _():
            nchunk(w + _NW)

        pltpu.sync_copy(cnt_e, out_cnts.at[w, 0])
        pltpu.sync_copy(cnt_v, out_cnts.at[w, 1])

        plsc.subcore_barrier()

        @pl.when(sid == 0)
        def _():
            pltpu.sync_copy(acc_e, out_sums.at[cid, 0])
            pltpu.sync_copy(acc_v, out_sums.at[cid, 1])

    return k(src, edge_attr, batch, x, zeros)


def _tc_finalize(psums, pcnts, state,
                 W1, b1, g1, be1, W2, b2, g2, be2, W3, b3, g3, be3):
    def body(ps_ref, pc_ref, st_ref,
             w1_ref, b1_ref, g1_ref, be1_ref,
             w2_ref, b2_ref, g2_ref, be2_ref,
             w3_ref, b3_ref, g3_ref, be3_ref, out_ref):
        ps = ps_ref[...]
        sums_e = ps[0, 0] + ps[1, 0]
        sums_v = ps[0, 1] + ps[1, 1]
        cnt = jnp.sum(pc_ref[...], axis=0)
        rec_e = jnp.clip(cnt[0], 1.0, None)[:, None]
        rec_v = jnp.clip(cnt[1], 1.0, None)[:, None]
        u_e = sums_e / rec_e
        u_v = sums_v / rec_v
        comb = jnp.concatenate([u_e, u_v, st_ref[...]], axis=1)

        def dense(h, w_ref, b_ref):
            return lax.dot_general(
                h, w_ref[...], (((1,), (1,)), ((), ())),
                preferred_element_type=jnp.float32) + b_ref[...][None, :]

        def bn(h, g_ref, be_ref):
            mean = jnp.mean(h, axis=0)
            var = jnp.mean((h - mean[None, :]) ** 2, axis=0)
            return (h - mean[None, :]) * (g_ref[...][None, :]
                                          * lax.rsqrt(var + _EPS)) + be_ref[...][None, :]

        h = bn(jax.nn.relu(dense(comb, w1_ref, b1_ref)), g1_ref, be1_ref)
        h = bn(jax.nn.relu(dense(h, w2_ref, b2_ref)), g2_ref, be2_ref)
        h = bn(dense(h, w3_ref, b3_ref), g3_ref, be3_ref)
        out_ref[...] = h

    return pl.pallas_call(
        body,
        out_shape=jax.ShapeDtypeStruct((_N_GRAPHS, _DIM), jnp.float32),
    )(psums, pcnts, state, W1, b1, g1, be1, W2, b2, g2, be2, W3, b3, g3, be3)


def kernel(x, edge_index, edge_attr, state, batch,
           W1, b1, g1, be1, W2, b2, g2, be2, W3, b3, g3, be3):
    src = edge_index[0]
    zeros = jnp.zeros((_N_GRAPHS, _DIM), jnp.float32)
    psums, pcnts = _sc_segment_sums(src, edge_attr, batch, x, zeros)
    return _tc_finalize(psums, pcnts, state,
                        W1, b1, g1, be1, W2, b2, g2, be2, W3, b3, g3, be3)


# speedup vs baseline: 17.7105x; 17.7105x over previous
import functools

import jax
import jax.numpy as jnp
from jax import lax
from jax.experimental import pallas as pl
from jax.experimental.pallas import tpu as pltpu
from jax.experimental.pallas import tpu_sc as plsc

_DIM = 32
_N_NODES = 100000
_N_EDGES = 1600000
_N_GRAPHS = 256
_EPS = 1e-5

_NC = 2
_NS = 16
_NW = _NC * _NS
_CH = 2000
_E_PER_W = _N_EDGES // _NW
_E_CHUNKS = _E_PER_W // _CH
_NODE_CHUNKS = _N_NODES // _CH


def _sc_segment_sums(src, edge_attr, batch, x, zeros):
    mesh = plsc.VectorSubcoreMesh(core_axis_name="c", subcore_axis_name="s")

    @functools.partial(
        pl.kernel,
        mesh=mesh,
        compiler_params=pltpu.CompilerParams(
            needs_layout_passes=False, use_tc_tiling_on_sc=False),
        out_type=[
            jax.ShapeDtypeStruct((_NC, 2, _N_GRAPHS, _DIM), jnp.float32),
            jax.ShapeDtypeStruct((_NW, 2, _N_GRAPHS), jnp.float32),
        ],
        scratch_types=[
            pltpu.VMEM((_CH,), jnp.int32),
            pltpu.VMEM((_CH,), jnp.int32),
            pltpu.VMEM((_CH, _DIM), jnp.float32),
            pltpu.VMEM((_N_GRAPHS,), jnp.float32),
            pltpu.VMEM((_N_GRAPHS,), jnp.float32),
            pltpu.VMEM_SHARED((_N_GRAPHS, _DIM), jnp.float32),
            pltpu.VMEM_SHARED((_N_GRAPHS, _DIM), jnp.float32),
            pltpu.SemaphoreType.DMA,
        ],
    )
    def k(src_hbm, attr_hbm, batch_hbm, x_hbm, zeros_hbm,
          out_sums, out_cnts,
          src_v, g_v, rows_v, cnt_e, cnt_v, acc_e, acc_v, sem):
        cid = lax.axis_index("c")
        sid = lax.axis_index("s")
        w = cid * _NS + sid

        @pl.when(sid == 0)
        def _():
            pltpu.sync_copy(zeros_hbm, acc_e)
            pltpu.sync_copy(zeros_hbm, acc_v)

        zero16 = jnp.zeros((16,), jnp.float32)

        def zinit(i, carry):
            cnt_e[pl.ds(i * 16, 16)] = zero16
            cnt_v[pl.ds(i * 16, 16)] = zero16
            return carry

        lax.fori_loop(0, _N_GRAPHS // 16, zinit, 0)

        plsc.subcore_barrier()

        ones16 = jnp.ones((16,), jnp.float32)

        def hist(cref):
            def body(j, carry):
                gv = g_v[pl.ds(j * 16, 16)]
                plsc.addupdate_scatter(cref, [gv], ones16)
                return carry
            lax.fori_loop(0, _CH // 16, body, 0)

        ebase = w * _E_PER_W

        def echunk(c, carry):
            off = ebase + c * _CH
            pltpu.sync_copy(src_hbm.at[pl.ds(off, _CH)], src_v)
            pltpu.async_copy(batch_hbm.at[src_v], g_v, sem).wait()
            pltpu.sync_copy(attr_hbm.at[pl.ds(off, _CH)], rows_v)
            pltpu.sync_copy(rows_v, acc_e.at[g_v], add=True)
            hist(cnt_e)
            return carry

        lax.fori_loop(0, _E_CHUNKS, echunk, 0)

        def nchunk(chunk):
            off = chunk * _CH
            pltpu.sync_copy(batch_hbm.at[pl.ds(off, _CH)], g_v)
            pltpu.sync_copy(x_hbm.at[pl.ds(off, _CH)], rows_v)
            pltpu.sync_copy(rows_v, acc_v.at[g_v], add=True)
            hist(cnt_v)

        nchunk(w)

        @pl.when(w + _NW < _NODE_CHUNKS)
        def _():
            nchunk(w + _NW)

        pltpu.sync_copy(cnt_e, out_cnts.at[w, 0])
        pltpu.sync_copy(cnt_v, out_cnts.at[w, 1])

        plsc.subcore_barrier()

        @pl.when(sid == 0)
        def _():
            pltpu.sync_copy(acc_e, out_sums.at[cid, 0])
            pltpu.sync_copy(acc_v, out_sums.at[cid, 1])

    return k(src, edge_attr, batch, x, zeros)


def _tc_finalize(psums, pcnts, state,
                 W1, b1, g1, be1, W2, b2, g2, be2, W3, b3, g3, be3):
    def body(ps_ref, pc_ref, st_ref,
             w1_ref, b1_ref, g1_ref, be1_ref,
             w2_ref, b2_ref, g2_ref, be2_ref,
             w3_ref, b3_ref, g3_ref, be3_ref, out_ref):
        ps = ps_ref[...]
        sums_e = ps[0, 0] + ps[1, 0]
        sums_v = ps[0, 1] + ps[1, 1]
        cnt = jnp.sum(pc_ref[...], axis=0)
        rec_e = jnp.clip(cnt[0], 1.0, None)[:, None]
        rec_v = jnp.clip(cnt[1], 1.0, None)[:, None]
        u_e = sums_e / rec_e
        u_v = sums_v / rec_v
        comb = jnp.concatenate([u_e, u_v, st_ref[...]], axis=1)

        def dense(h, w_ref, b_ref):
            return lax.dot_general(
                h, w_ref[...], (((1,), (1,)), ((), ())),
                preferred_element_type=jnp.float32) + b_ref[...][None, :]

        def bn(h, g_ref, be_ref):
            mean = jnp.mean(h, axis=0)
            var = jnp.mean((h - mean[None, :]) ** 2, axis=0)
            return (h - mean[None, :]) * (g_ref[...][None, :]
                                          * lax.rsqrt(var + _EPS)) + be_ref[...][None, :]

        h = bn(jax.nn.relu(dense(comb, w1_ref, b1_ref)), g1_ref, be1_ref)
        h = bn(jax.nn.relu(dense(h, w2_ref, b2_ref)), g2_ref, be2_ref)
        h = bn(dense(h, w3_ref, b3_ref), g3_ref, be3_ref)
        out_ref[...] = h

    return pl.pallas_call(
        body,
        out_shape=jax.ShapeDtypeStruct((_N_GRAPHS, _DIM), jnp.float32),
    )(psums, pcnts, state, W1, b1, g1, be1, W2, b2, g2, be2, W3, b3, g3, be3)


def kernel(x, edge_index, edge_attr, state, batch,
           W1, b1, g1, be1, W2, b2, g2, be2, W3, b3, g3, be3):
    src = edge_index[0]
    zeros = jnp.zeros((_N_GRAPHS, _DIM), jnp.float32)
    psums, pcnts = _sc_segment_sums(src, edge_attr, batch, x, zeros)
    return _tc_finalize(psums, pcnts, state,
                        W1, b1, g1, be1, W2, b2, g2, be2, W3, b3, g3, be3)


# speedup vs baseline: 18.6701x; 1.0542x over previous
import functools

import jax
import jax.numpy as jnp
from jax import lax
from jax.experimental import pallas as pl
from jax.experimental.pallas import tpu as pltpu
from jax.experimental.pallas import tpu_sc as plsc

_DIM = 32
_N_NODES = 100000
_N_EDGES = 1600000
_N_GRAPHS = 256
_EPS = 1e-5

_NC = 2
_NS = 16
_NW = _NC * _NS
_CH = 1000
_E_PER_W = _N_EDGES // _NW
_E_CHUNKS = _E_PER_W // _CH
_NODE_CHUNKS = _N_NODES // _CH
_FULL_GROUPS = _CH // 16
_REM = _CH - _FULL_GROUPS * 16


def _sc_segment_sums(edge_index, edge_attr, batch, x, zeros):
    mesh = plsc.VectorSubcoreMesh(core_axis_name="c", subcore_axis_name="s")

    @functools.partial(
        pl.kernel,
        mesh=mesh,
        compiler_params=pltpu.CompilerParams(
            needs_layout_passes=False, use_tc_tiling_on_sc=False),
        out_type=[
            jax.ShapeDtypeStruct((_NC, 2, _N_GRAPHS, _DIM), jnp.float32),
            jax.ShapeDtypeStruct((_NW, 2, _N_GRAPHS), jnp.float32),
        ],
        scratch_types=[
            pltpu.VMEM((2, _CH), jnp.int32),
            pltpu.VMEM((2, _CH), jnp.int32),
            pltpu.VMEM((2, _CH, _DIM), jnp.float32),
            pltpu.VMEM((_N_GRAPHS,), jnp.float32),
            pltpu.VMEM((_N_GRAPHS,), jnp.float32),
            pltpu.VMEM_SHARED((_N_GRAPHS, _DIM), jnp.float32),
            pltpu.VMEM_SHARED((_N_GRAPHS, _DIM), jnp.float32),
            pltpu.SemaphoreType.DMA,
            pltpu.SemaphoreType.DMA,
            pltpu.SemaphoreType.DMA,
            pltpu.SemaphoreType.DMA,
            pltpu.SemaphoreType.DMA,
            pltpu.SemaphoreType.DMA,
        ],
    )
    def k(eidx_hbm, attr_hbm, batch_hbm, x_hbm, zeros_hbm,
          out_sums, out_cnts,
          src_v, g_v, rows_v, cnt_e, cnt_v, acc_e, acc_v,
          sem_s0, sem_s1, sem_a0, sem_a1, sem_g0, sem_g1):
        sem_src = (sem_s0, sem_s1)
        sem_attr = (sem_a0, sem_a1)
        sem_g = (sem_g0, sem_g1)
        cid = lax.axis_index("c")
        sid = lax.axis_index("s")
        w = cid * _NS + sid

        @pl.when(sid == 0)
        def _():
            pltpu.sync_copy(zeros_hbm, acc_e)
            pltpu.sync_copy(zeros_hbm, acc_v)

        zero16 = jnp.zeros((16,), jnp.float32)

        def zinit(i, carry):
            cnt_e[pl.ds(i * 16, 16)] = zero16
            cnt_v[pl.ds(i * 16, 16)] = zero16
            return carry

        lax.fori_loop(0, _N_GRAPHS // 16, zinit, 0)

        plsc.subcore_barrier()

        ones16 = jnp.ones((16,), jnp.float32)
        tail_mask = lax.iota(jnp.int32, 16) >= (16 - _REM)

        def hist(cref, b):
            def body(j, carry):
                gv = g_v[b, pl.ds(j * 16, 16)]
                plsc.addupdate_scatter(cref, [gv], ones16)
                return carry
            lax.fori_loop(0, _FULL_GROUPS, body, 0)
            if _REM:
                gv = g_v[b, pl.ds(_CH - 16, 16)]
                plsc.addupdate_scatter(cref, [gv], ones16, mask=tail_mask)

        ebase = w * _E_PER_W

        def issue_loads(c):
            b = c % 2
            off = ebase + c * _CH
            hs = pltpu.async_copy(
                eidx_hbm.at[0, pl.ds(off, _CH)], src_v.at[b], sem_src[b])
            ha = pltpu.async_copy(
                attr_hbm.at[pl.ds(off, _CH)], rows_v.at[b], sem_attr[b])
            return hs, ha

        handles = issue_loads(0)
        for c in range(_E_CHUNKS):
            b = c % 2
            hs, ha = handles
            hs.wait()
            hg = pltpu.async_copy(
                batch_hbm.at[src_v.at[b]], g_v.at[b], sem_g[b])
            if c + 1 < _E_CHUNKS:
                handles = issue_loads(c + 1)
            ha.wait()
            hg.wait()
            pltpu.sync_copy(rows_v.at[b], acc_e.at[g_v.at[b]], add=True)
            hist(cnt_e, b)

        def nchunk(chunk):
            off = chunk * _CH
            pltpu.sync_copy(batch_hbm.at[pl.ds(off, _CH)], g_v.at[0])
            pltpu.sync_copy(x_hbm.at[pl.ds(off, _CH)], rows_v.at[0])
            pltpu.sync_copy(rows_v.at[0], acc_v.at[g_v.at[0]], add=True)
            hist(cnt_v, 0)

        for kk in range(4):
            if (kk + 1) * _NW <= _NODE_CHUNKS:
                nchunk(w + kk * _NW)
            else:
                @pl.when(w + kk * _NW < _NODE_CHUNKS)
                def _():
                    nchunk(w + kk * _NW)

        pltpu.sync_copy(cnt_e, out_cnts.at[w, 0])
        pltpu.sync_copy(cnt_v, out_cnts.at[w, 1])

        plsc.subcore_barrier()

        @pl.when(sid == 0)
        def _():
            pltpu.sync_copy(acc_e, out_sums.at[cid, 0])
            pltpu.sync_copy(acc_v, out_sums.at[cid, 1])

    return k(edge_index, edge_attr, batch, x, zeros)


def _tc_finalize(psums, pcnts, state,
                 W1, b1, g1, be1, W2, b2, g2, be2, W3, b3, g3, be3):
    def body(ps_ref, pc_ref, st_ref,
             w1_ref, b1_ref, g1_ref, be1_ref,
             w2_ref, b2_ref, g2_ref, be2_ref,
             w3_ref, b3_ref, g3_ref, be3_ref, out_ref):
        ps = ps_ref[...]
        sums_e = ps[0, 0] + ps[1, 0]
        sums_v = ps[0, 1] + ps[1, 1]
        cnt = jnp.sum(pc_ref[...], axis=0)
        rec_e = jnp.clip(cnt[0], 1.0, None)[:, None]
        rec_v = jnp.clip(cnt[1], 1.0, None)[:, None]
        u_e = sums_e / rec_e
        u_v = sums_v / rec_v
        comb = jnp.concatenate([u_e, u_v, st_ref[...]], axis=1)

        def dense(h, w_ref, b_ref):
            return lax.dot_general(
                h, w_ref[...], (((1,), (1,)), ((), ())),
                preferred_element_type=jnp.float32) + b_ref[...][None, :]

        def bn(h, g_ref, be_ref):
            mean = jnp.mean(h, axis=0)
            var = jnp.mean((h - mean[None, :]) ** 2, axis=0)
            return (h - mean[None, :]) * (g_ref[...][None, :]
                                          * lax.rsqrt(var + _EPS)) + be_ref[...][None, :]

        h = bn(jax.nn.relu(dense(comb, w1_ref, b1_ref)), g1_ref, be1_ref)
        h = bn(jax.nn.relu(dense(h, w2_ref, b2_ref)), g2_ref, be2_ref)
        h = bn(dense(h, w3_ref, b3_ref), g3_ref, be3_ref)
        out_ref[...] = h

    return pl.pallas_call(
        body,
        out_shape=jax.ShapeDtypeStruct((_N_GRAPHS, _DIM), jnp.float32),
    )(psums, pcnts, state, W1, b1, g1, be1, W2, b2, g2, be2, W3, b3, g3, be3)


def kernel(x, edge_index, edge_attr, state, batch,
           W1, b1, g1, be1, W2, b2, g2, be2, W3, b3, g3, be3):
    zeros = jnp.zeros((_N_GRAPHS, _DIM), jnp.float32)
    psums, pcnts = _sc_segment_sums(edge_index, edge_attr, batch, x, zeros)
    return _tc_finalize(psums, pcnts, state,
                        W1, b1, g1, be1, W2, b2, g2, be2, W3, b3, g3, be3)
